# baseline (device time: 20784 ns/iter reference)
import jax
import jax.numpy as jnp
from jax import lax
from jax.experimental import pallas as pl
from jax.experimental.pallas import tpu as pltpu

CHUNK_ROWS = [128] * 6 + [80, 64, 48, 32, 16, 8, 8]
N_CHUNKS = len(CHUNK_ROWS)
CHUNK_OFF = [sum(CHUNK_ROWS[:k]) for k in range(N_CHUNKS)]


def kernel(x):
    m, n = x.shape
    half = m // 2
    assert sum(CHUNK_ROWS) == half

    my_y_out = lax.axis_index("y")
    x_half = lax.dynamic_slice_in_dim(x, my_y_out * half, half, axis=0)

    def body(x_ref, out_ref, send_ref, recv_ref,
             send_sems1, recv_sems1, send_sems2, recv_sems2):
        my_x = lax.axis_index("x")
        my_y = lax.axis_index("y")
        x_nbr = (1 - my_x, my_y)
        y_nbr = (my_x, 1 - my_y)
        row0 = my_y * half

        send_ref[:, :] = x_ref[:, :].astype(jnp.bfloat16)

        barrier_sem = pltpu.get_barrier_semaphore()
        for nbr in (x_nbr, y_nbr):
            pl.semaphore_signal(
                barrier_sem, inc=1,
                device_id=nbr, device_id_type=pl.DeviceIdType.MESH,
            )
        pl.semaphore_wait(barrier_sem, 2)

        rdma1 = []
        for k in range(N_CHUNKS):
            rows_loc = pl.ds(CHUNK_OFF[k], CHUNK_ROWS[k])
            r = pltpu.make_async_remote_copy(
                src_ref=send_ref.at[rows_loc, :],
                dst_ref=recv_ref.at[rows_loc, :],
                send_sem=send_sems1.at[k],
                recv_sem=recv_sems1.at[k],
                device_id=x_nbr,
                device_id_type=pl.DeviceIdType.MESH,
            )
            r.start()
            rdma1.append(r)

        rdma2 = []
        for k in range(N_CHUNKS):
            rdma1[k].wait_recv()
            rows_loc = pl.ds(CHUNK_OFF[k], CHUNK_ROWS[k])
            rows = pl.ds(row0 + CHUNK_OFF[k], CHUNK_ROWS[k])
            out_ref[rows, :] = send_ref[rows_loc, :] + recv_ref[rows_loc, :]
            r = pltpu.make_async_remote_copy(
                src_ref=out_ref.at[rows, :],
                dst_ref=out_ref.at[rows, :],
                send_sem=send_sems2.at[k],
                recv_sem=recv_sems2.at[k],
                device_id=y_nbr,
                device_id_type=pl.DeviceIdType.MESH,
            )
            r.start()
            rdma2.append(r)

        for k in range(N_CHUNKS):
            rdma1[k].wait_send()
            rdma2[k].wait()

    return pl.pallas_call(
        body,
        out_shape=jax.ShapeDtypeStruct((m, n), jnp.bfloat16),
        in_specs=[pl.BlockSpec(memory_space=pltpu.VMEM)],
        out_specs=pl.BlockSpec(memory_space=pltpu.VMEM),
        scratch_shapes=[
            pltpu.VMEM((half, n), jnp.bfloat16),
            pltpu.VMEM((half, n), jnp.bfloat16),
            pltpu.SemaphoreType.DMA((N_CHUNKS,)),
            pltpu.SemaphoreType.DMA((N_CHUNKS,)),
            pltpu.SemaphoreType.DMA((N_CHUNKS,)),
            pltpu.SemaphoreType.DMA((N_CHUNKS,)),
        ],
        compiler_params=pltpu.CompilerParams(collective_id=0),
    )(x_half)


# device time: 20189 ns/iter; 1.0295x vs baseline; 1.0295x over previous
import jax
import jax.numpy as jnp
from jax import lax
from jax.experimental import pallas as pl
from jax.experimental.pallas import tpu as pltpu

CHUNK_ROWS = [64] * 16
N_CHUNKS = len(CHUNK_ROWS)
CHUNK_OFF = [sum(CHUNK_ROWS[:k]) for k in range(N_CHUNKS)]


def kernel(x):
    m, n = x.shape
    half = m // 2
    assert sum(CHUNK_ROWS) == half

    my_y_out = lax.axis_index("y")
    x_half = lax.dynamic_slice_in_dim(x, my_y_out * half, half, axis=0)

    def body(x_ref, out_ref, send_ref, recv_ref,
             send_sems1, recv_sems1, send_sems2, recv_sems2):
        my_x = lax.axis_index("x")
        my_y = lax.axis_index("y")
        x_nbr = (1 - my_x, my_y)
        y_nbr = (my_x, 1 - my_y)
        row0 = my_y * half

        send_ref[:, :] = x_ref[:, :].astype(jnp.bfloat16)

        barrier_sem = pltpu.get_barrier_semaphore()
        for nbr in (x_nbr, y_nbr):
            pl.semaphore_signal(
                barrier_sem, inc=1,
                device_id=nbr, device_id_type=pl.DeviceIdType.MESH,
            )
        pl.semaphore_wait(barrier_sem, 2)

        rdma1 = []
        for k in range(N_CHUNKS):
            rows_loc = pl.ds(CHUNK_OFF[k], CHUNK_ROWS[k])
            r = pltpu.make_async_remote_copy(
                src_ref=send_ref.at[rows_loc, :],
                dst_ref=recv_ref.at[rows_loc, :],
                send_sem=send_sems1.at[k],
                recv_sem=recv_sems1.at[k],
                device_id=x_nbr,
                device_id_type=pl.DeviceIdType.MESH,
            )
            r.start()
            rdma1.append(r)

        rdma2 = []
        for k in range(N_CHUNKS):
            rdma1[k].wait_recv()
            rows_loc = pl.ds(CHUNK_OFF[k], CHUNK_ROWS[k])
            rows = pl.ds(row0 + CHUNK_OFF[k], CHUNK_ROWS[k])
            out_ref[rows, :] = send_ref[rows_loc, :] + recv_ref[rows_loc, :]
            r = pltpu.make_async_remote_copy(
                src_ref=out_ref.at[rows, :],
                dst_ref=out_ref.at[rows, :],
                send_sem=send_sems2.at[k],
                recv_sem=recv_sems2.at[k],
                device_id=y_nbr,
                device_id_type=pl.DeviceIdType.MESH,
            )
            r.start()
            rdma2.append(r)

        for k in range(N_CHUNKS):
            rdma1[k].wait_send()
            rdma2[k].wait()

    return pl.pallas_call(
        body,
        out_shape=jax.ShapeDtypeStruct((m, n), jnp.bfloat16),
        in_specs=[pl.BlockSpec(memory_space=pltpu.VMEM)],
        out_specs=pl.BlockSpec(memory_space=pltpu.VMEM),
        scratch_shapes=[
            pltpu.VMEM((half, n), jnp.bfloat16),
            pltpu.VMEM((half, n), jnp.bfloat16),
            pltpu.SemaphoreType.DMA((N_CHUNKS,)),
            pltpu.SemaphoreType.DMA((N_CHUNKS,)),
            pltpu.SemaphoreType.DMA((N_CHUNKS,)),
            pltpu.SemaphoreType.DMA((N_CHUNKS,)),
        ],
        compiler_params=pltpu.CompilerParams(collective_id=0),
    )(x_half)
